# Initial kernel scaffold; baseline (speedup 1.0000x reference)
import jax
import jax.numpy as jnp
from jax.experimental import pallas as pl


def kernel(inputs, reference_points):
    raise NotImplementedError("write your pallas kernel here")


# speedup vs baseline: 5.6590x; 5.6590x over previous
import functools

import jax
import jax.numpy as jnp
from jax import lax
from jax.experimental import pallas as pl
from jax.experimental.pallas import tpu as pltpu

K = 5
NUM_REF = 100
D = 128
B = 16384
NPAD = 128
BLOCK_B = 512


def _knn_body(x_ref, w_ref, rn_ref, out_ref):
    x = x_ref[...]
    w = w_ref[...]
    s = jnp.dot(x, w, preferred_element_type=jnp.float32,
                precision=lax.Precision.HIGHEST)
    s = s + rn_ref[...]
    iota = lax.broadcasted_iota(jnp.int32, s.shape, 1)
    cols = []
    for _ in range(K):
        m = jnp.min(s, axis=1, keepdims=True)
        idx = jnp.min(jnp.where(s == m, iota, jnp.int32(2**30)), axis=1)
        cols.append(idx)
        s = jnp.where(iota == idx[:, None], jnp.float32(jnp.inf), s)
    out_ref[...] = jnp.stack(cols, axis=1)


@jax.jit
def kernel(inputs, reference_points):
    w = jnp.zeros((D, NPAD), jnp.float32)
    w = w.at[:, :NUM_REF].set(-2.0 * reference_points.T)
    rn = jnp.full((1, NPAD), 3e38, jnp.float32)
    rn = rn.at[0, :NUM_REF].set(jnp.sum(reference_points * reference_points,
                                        axis=1))
    grid = B // BLOCK_B
    return pl.pallas_call(
        _knn_body,
        grid=(grid,),
        in_specs=[
            pl.BlockSpec((BLOCK_B, D), lambda i: (i, 0)),
            pl.BlockSpec((D, NPAD), lambda i: (0, 0)),
            pl.BlockSpec((1, NPAD), lambda i: (0, 0)),
        ],
        out_specs=pl.BlockSpec((BLOCK_B, K), lambda i: (i, 0)),
        out_shape=jax.ShapeDtypeStruct((B, K), jnp.int32),
    )(inputs, w, rn)


# speedup vs baseline: 7.9627x; 1.4071x over previous
import functools

import jax
import jax.numpy as jnp
from jax import lax
from jax.experimental import pallas as pl
from jax.experimental.pallas import tpu as pltpu

K = 5
NUM_REF = 100
D = 128
B = 16384
NPAD = 128
BLOCK_B = 512


def _knn_body(x_ref, w_ref, rn_ref, out_ref):
    x = x_ref[...]
    w = w_ref[...]
    s = jnp.dot(x, w, preferred_element_type=jnp.float32,
                precision=lax.Precision.HIGHEST)
    s = s + rn_ref[...]
    iota = lax.broadcasted_iota(jnp.int32, s.shape, 1).astype(jnp.float32)
    cols = []
    for _ in range(K):
        m = jnp.min(s, axis=1, keepdims=True)
        idx = jnp.min(jnp.where(s == m, iota, jnp.float32(3e38)), axis=1)
        cols.append(idx)
        s = jnp.where(iota == idx[:, None], jnp.float32(jnp.inf), s)
    out_ref[...] = jnp.stack(cols, axis=1).astype(jnp.int32)


@jax.jit
def kernel(inputs, reference_points):
    w = jnp.zeros((D, NPAD), jnp.float32)
    w = w.at[:, :NUM_REF].set(-2.0 * reference_points.T)
    rn = jnp.full((1, NPAD), 3e38, jnp.float32)
    rn = rn.at[0, :NUM_REF].set(jnp.sum(reference_points * reference_points,
                                        axis=1))
    grid = B // BLOCK_B
    return pl.pallas_call(
        _knn_body,
        grid=(grid,),
        in_specs=[
            pl.BlockSpec((BLOCK_B, D), lambda i: (i, 0)),
            pl.BlockSpec((D, NPAD), lambda i: (0, 0)),
            pl.BlockSpec((1, NPAD), lambda i: (0, 0)),
        ],
        out_specs=pl.BlockSpec((BLOCK_B, K), lambda i: (i, 0)),
        out_shape=jax.ShapeDtypeStruct((B, K), jnp.int32),
    )(inputs, w, rn)


# speedup vs baseline: 10.4950x; 1.3180x over previous
import functools

import jax
import jax.numpy as jnp
from jax import lax
from jax.experimental import pallas as pl
from jax.experimental.pallas import tpu as pltpu

K = 5
NUM_REF = 100
D = 128
B = 16384
NPAD = 128
BLOCK_B = 2048


def _knn_body(x_ref, r_ref, rn_ref, out_ref):
    x = x_ref[...]
    r = r_ref[...]
    d = lax.dot_general(x, r, (((1,), (1,)), ((), ())),
                        preferred_element_type=jnp.float32,
                        precision=lax.Precision.HIGHEST)
    iota = lax.broadcasted_iota(jnp.int32, (BLOCK_B, NPAD), 1).astype(
        jnp.float32)
    pad = jnp.pad(d, ((0, 0), (0, NPAD - NUM_REF)))
    s = jnp.where(iota < float(NUM_REF), rn_ref[...] - 2.0 * pad,
                  jnp.float32(3e38))
    cols = []
    for _ in range(K):
        m = jnp.min(s, axis=1, keepdims=True)
        is_min = s == m
        idx = jnp.min(jnp.where(is_min, iota, jnp.float32(3e38)), axis=1)
        cols.append(idx)
        s = jnp.where(is_min, jnp.float32(jnp.inf), s)
    out_ref[...] = jnp.stack(cols, axis=1).astype(jnp.int32)


@jax.jit
def kernel(inputs, reference_points):
    rn = jnp.pad(jnp.sum(reference_points * reference_points, axis=1),
                 (0, NPAD - NUM_REF))[None, :]
    grid = B // BLOCK_B
    return pl.pallas_call(
        _knn_body,
        grid=(grid,),
        in_specs=[
            pl.BlockSpec((BLOCK_B, D), lambda i: (i, 0)),
            pl.BlockSpec((NUM_REF, D), lambda i: (0, 0)),
            pl.BlockSpec((1, NPAD), lambda i: (0, 0)),
        ],
        out_specs=pl.BlockSpec((BLOCK_B, K), lambda i: (i, 0)),
        out_shape=jax.ShapeDtypeStruct((B, K), jnp.int32),
    )(inputs, reference_points, rn)
